# Initial kernel scaffold; baseline (speedup 1.0000x reference)
import jax
import jax.numpy as jnp
from jax.experimental import pallas as pl


def kernel(L_values_real, L_values_imag, x_real, x_imag, L_indices):
    raise NotImplementedError("write your pallas kernel here")


# speedup vs baseline: 3.4922x; 3.4922x over previous
import functools

import jax
import jax.numpy as jnp
from jax import lax
from jax.experimental import pallas as pl
from jax.experimental.pallas import tpu as pltpu
from jax.experimental.pallas import tpu_sc as plsc

N = 10000
E = 320000
D = 128
H = D // 2
NC = 2
NS = 16
EPW = E // NS
C = 80
NCHUNK = EPW // C
ZR = 250
ZCOPIES = (2 * N) // NS // ZR


def _sc_body(rows_hbm, cols_hbm, vr_hbm, vi_hbm, xr2_hbm, xi2_hbm,
             out_hbm,
             acc, rows_v, ridx_v, cols_v, gidx_v, vr_v, vi_v,
             xr_g, xi_g, og_r, og_i, zbuf, sem, sem2):
    cid = lax.axis_index("c")
    sid = lax.axis_index("s")

    def _zero_zbuf(r, _):
        z = jnp.zeros((16,), jnp.float32)
        for j in range(4):
            zbuf[r, pl.ds(j * 16, 16)] = z
        return 0

    lax.fori_loop(0, ZR, _zero_zbuf, 0)
    for t in range(ZCOPIES):
        pltpu.sync_copy(zbuf, acc.at[pl.ds(sid * (2 * N // NS) + t * ZR, ZR)])
    plsc.subcore_barrier()

    base0 = sid * EPW

    def _chunk(g, _):
        base = base0 + g * C
        pltpu.sync_copy(rows_hbm.at[pl.ds(base, C)], rows_v)
        pltpu.sync_copy(cols_hbm.at[pl.ds(base, C)], cols_v)
        pltpu.sync_copy(vr_hbm.at[pl.ds(base, C)], vr_v)
        pltpu.sync_copy(vi_hbm.at[pl.ds(base, C)], vi_v)

        def _idx(i, _):
            cw = cols_v[pl.ds(i * 16, 16)]
            gidx_v[pl.ds(i * 16, 16)] = cw * 2 + cid
            rw = rows_v[pl.ds(i * 16, 16)]
            ridx_v[pl.ds(i * 16, 16)] = rw + N
            return 0

        lax.fori_loop(0, C // 16, _idx, 0)

        cp1 = pltpu.async_copy(xr2_hbm.at[gidx_v], xr_g, sem)
        cp2 = pltpu.async_copy(xi2_hbm.at[gidx_v], xi_g, sem2)
        cp1.wait()
        cp2.wait()

        def _edge(k, _):
            kk = jnp.full((16,), k, jnp.int32)
            av = plsc.load_gather(vr_v, [kk])
            bv = plsc.load_gather(vi_v, [kk])
            for j in range(4):
                xrj = xr_g[k, pl.ds(j * 16, 16)]
                xij = xi_g[k, pl.ds(j * 16, 16)]
                og_r[k, pl.ds(j * 16, 16)] = av * xrj - bv * xij
                og_i[k, pl.ds(j * 16, 16)] = av * xij + bv * xrj
            return 0

        lax.fori_loop(0, C, _edge, 0)

        pltpu.sync_copy(og_r, acc.at[rows_v], add=True)
        pltpu.sync_copy(og_i, acc.at[ridx_v], add=True)
        return 0

    lax.fori_loop(0, NCHUNK, _chunk, 0)
    plsc.subcore_barrier()

    rpt = N // NS
    for q in range(2):
        pltpu.sync_copy(
            acc.at[pl.ds(q * N + sid * rpt, rpt)],
            out_hbm.at[q, pl.ds(sid * rpt, rpt), pl.ds(cid * H, H)],
        )


@jax.jit
def kernel(L_values_real, L_values_imag, x_real, x_imag, L_indices):
    rows = L_indices[0]
    cols = L_indices[1]
    xr2 = x_real.reshape(2 * N, H)
    xi2 = x_imag.reshape(2 * N, H)

    mesh = plsc.VectorSubcoreMesh(
        core_axis_name="c", subcore_axis_name="s", num_cores=NC,
        num_subcores=NS)
    f = pl.kernel(
        _sc_body,
        out_type=jax.ShapeDtypeStruct((2, N, D), jnp.float32),
        mesh=mesh,
        compiler_params=pltpu.CompilerParams(use_tc_tiling_on_sc=False,
                                             needs_layout_passes=False),
        scratch_types=[
            pltpu.VMEM_SHARED((2 * N, H), jnp.float32),
            pltpu.VMEM((C,), jnp.int32),
            pltpu.VMEM((C,), jnp.int32),
            pltpu.VMEM((C,), jnp.int32),
            pltpu.VMEM((C,), jnp.int32),
            pltpu.VMEM((C,), jnp.float32),
            pltpu.VMEM((C,), jnp.float32),
            pltpu.VMEM((C, H), jnp.float32),
            pltpu.VMEM((C, H), jnp.float32),
            pltpu.VMEM((C, H), jnp.float32),
            pltpu.VMEM((C, H), jnp.float32),
            pltpu.VMEM((ZR, H), jnp.float32),
            pltpu.SemaphoreType.DMA,
            pltpu.SemaphoreType.DMA,
        ],
    )
    return f(rows, cols, L_values_real, L_values_imag, xr2, xi2)


# speedup vs baseline: 5.7884x; 1.6575x over previous
import jax
import jax.numpy as jnp
from jax import lax
from jax.experimental import pallas as pl
from jax.experimental.pallas import tpu as pltpu
from jax.experimental.pallas import tpu_sc as plsc

N = 10000
E = 320000
D = 128
H = D // 2
NC = 2
NS = 16
EPW = E // NS
NSUP = 10
S = EPW // NSUP
C = 128
SP = 2048
NCHN = SP // C
NPAIR = NCHN // 2
ZFULL = (2 * N // NS) // C
ZREM = (2 * N // NS) % C


def _sc_body(rows4_hbm, cols4_hbm, vr4_hbm, vi4_hbm, xr2_hbm, xi2_hbm,
             out_hbm,
             acc, rows2, ridx2, gidx2, vrv, viv,
             xg_r0, xg_i0, xg_r1, xg_i1, sem_g0, sem_g1):
    cid = lax.axis_index("c")
    sid = lax.axis_index("s")

    def _zero_buf(r, _):
        z = jnp.zeros((16,), jnp.float32)
        for j in range(4):
            xg_r0[r, pl.ds(j * 16, 16)] = z
        return 0

    lax.fori_loop(0, C, _zero_buf, 0)
    zbase = sid * (2 * N // NS)
    for t in range(ZFULL):
        pltpu.sync_copy(xg_r0, acc.at[pl.ds(zbase + t * C, C)])
    pltpu.sync_copy(xg_r0.at[pl.ds(0, ZREM)],
                    acc.at[pl.ds(zbase + ZFULL * C, ZREM)])
    plsc.subcore_barrier()

    def _start_gathers(jj, xr_buf, xi_buf, sem):
        pltpu.async_copy(xr2_hbm.at[gidx2.at[jj]], xr_buf, sem)
        pltpu.async_copy(xi2_hbm.at[gidx2.at[jj]], xi_buf, sem)

    def _wait_gathers(jj, xr_buf, xi_buf, sem):
        pltpu.make_async_copy(xr2_hbm.at[gidx2.at[jj]], xr_buf, sem).wait()
        pltpu.make_async_copy(xi2_hbm.at[gidx2.at[jj]], xi_buf, sem).wait()

    def _compute_chunk(jj, xr_buf, xi_buf):
        def _edge(k, _):
            kk = jnp.full((16,), jj * C + k, jnp.int32)
            av = plsc.load_gather(vrv, [kk])
            bv = plsc.load_gather(viv, [kk])
            for j in range(4):
                sl = pl.ds(j * 16, 16)
                xr = xr_buf[k, sl]
                xi = xi_buf[k, sl]
                xr_buf[k, sl] = av * xr - bv * xi
                xi_buf[k, sl] = av * xi + bv * xr
            return 0

        lax.fori_loop(0, C, _edge, 0, unroll=4)

    def _half(jj, cur, nxt, sem_cur, sem_nxt):
        @pl.when(jj + 1 < NCHN)
        def _():
            _start_gathers(jj + 1, nxt[0], nxt[1], sem_nxt)

        _wait_gathers(jj, cur[0], cur[1], sem_cur)
        _compute_chunk(jj, cur[0], cur[1])
        pltpu.sync_copy(cur[0], acc.at[rows2.at[jj]], add=True)
        pltpu.sync_copy(cur[1], acc.at[ridx2.at[jj]], add=True)

    def _super(s, _):
        pltpu.sync_copy(rows4_hbm.at[sid, s], rows2)
        pltpu.sync_copy(rows4_hbm.at[sid, s], ridx2)
        pltpu.sync_copy(cols4_hbm.at[sid, s], gidx2)
        pltpu.sync_copy(vr4_hbm.at[sid, s], vrv)
        pltpu.sync_copy(vi4_hbm.at[sid, s], viv)

        def _idx(jc, _):
            for i in range(C // 16):
                sl = pl.ds(i * 16, 16)
                ridx2[jc, sl] = ridx2[jc, sl] + N
                gidx2[jc, sl] = gidx2[jc, sl] * 2 + cid
            return 0

        lax.fori_loop(0, NCHN, _idx, 0)

        _start_gathers(0, xg_r0, xg_i0, sem_g0)

        def _pair(p, _):
            _half(2 * p, (xg_r0, xg_i0), (xg_r1, xg_i1), sem_g0, sem_g1)
            _half(2 * p + 1, (xg_r1, xg_i1), (xg_r0, xg_i0), sem_g1, sem_g0)
            return 0

        lax.fori_loop(0, NPAIR, _pair, 0)
        return 0

    lax.fori_loop(0, NSUP, _super, 0)
    plsc.subcore_barrier()

    rpt = N // NS
    for q in range(2):
        pltpu.sync_copy(
            acc.at[pl.ds(q * N + sid * rpt, rpt)],
            out_hbm.at[q, pl.ds(sid * rpt, rpt), pl.ds(cid * H, H)],
        )


@jax.jit
def kernel(L_values_real, L_values_imag, x_real, x_imag, L_indices):
    def _pad3(a):
        return jnp.pad(a.reshape(NS, NSUP, S), ((0, 0), (0, 0), (0, SP - S)))

    rows4 = _pad3(L_indices[0]).reshape(NS, NSUP, NCHN, C)
    cols4 = _pad3(L_indices[1]).reshape(NS, NSUP, NCHN, C)
    vr4 = _pad3(L_values_real)
    vi4 = _pad3(L_values_imag)
    xr2 = x_real.reshape(2 * N, H)
    xi2 = x_imag.reshape(2 * N, H)

    mesh = plsc.VectorSubcoreMesh(
        core_axis_name="c", subcore_axis_name="s", num_cores=NC,
        num_subcores=NS)
    f = pl.kernel(
        _sc_body,
        out_type=jax.ShapeDtypeStruct((2, N, D), jnp.float32),
        mesh=mesh,
        compiler_params=pltpu.CompilerParams(use_tc_tiling_on_sc=False,
                                             needs_layout_passes=False),
        scratch_types=[
            pltpu.VMEM_SHARED((2 * N, H), jnp.float32),
            pltpu.VMEM((NCHN, C), jnp.int32),
            pltpu.VMEM((NCHN, C), jnp.int32),
            pltpu.VMEM((NCHN, C), jnp.int32),
            pltpu.VMEM((SP,), jnp.float32),
            pltpu.VMEM((SP,), jnp.float32),
            pltpu.VMEM((C, H), jnp.float32),
            pltpu.VMEM((C, H), jnp.float32),
            pltpu.VMEM((C, H), jnp.float32),
            pltpu.VMEM((C, H), jnp.float32),
            pltpu.SemaphoreType.DMA,
            pltpu.SemaphoreType.DMA,
        ],
    )
    return f(rows4, cols4, vr4, vi4, xr2, xi2)


# speedup vs baseline: 7.9765x; 1.3780x over previous
import jax
import jax.numpy as jnp
from jax import lax
from jax.experimental import pallas as pl
from jax.experimental.pallas import tpu as pltpu
from jax.experimental.pallas import tpu_sc as plsc

N = 10000
E = 320000
D = 128
H = D // 2
NC = 2
NS = 16
EPW = E // NS
NSUP = 10
S = EPW // NSUP
C = 128
HC = C // 2
SP = 2048
NCHN = SP // C
NPAIR = NCHN // 2
RPT = N // NS
ZFULL = RPT // C
ZREM = RPT % C


def _sc_body(rows4_hbm, cols4_hbm, vr4_hbm, vi4_hbm, xc_hbm,
             out_hbm,
             acc, rows2v, gidx2, vrv, viv,
             xg0, xg1, sem_g0, sem_g1, sem_s0, sem_s1):
    cid = lax.axis_index("c")
    sid = lax.axis_index("s")

    def _zero_buf(r, _):
        z = jnp.zeros((16,), jnp.float32)
        for j in range(8):
            xg0[r, pl.ds(j * 16, 16)] = z
        return 0

    lax.fori_loop(0, C, _zero_buf, 0)
    zbase = sid * RPT
    for t in range(ZFULL):
        pltpu.sync_copy(xg0, acc.at[pl.ds(zbase + t * C, C)])
    pltpu.sync_copy(xg0.at[pl.ds(0, ZREM)],
                    acc.at[pl.ds(zbase + ZFULL * C, ZREM)])
    plsc.subcore_barrier()

    def _start_gather(jj, buf, sem):
        pltpu.async_copy(xc_hbm.at[gidx2.at[jj]], buf, sem)

    def _wait_gather(jj, buf, sem):
        pltpu.make_async_copy(xc_hbm.at[gidx2.at[jj]], buf, sem).wait()

    def _start_scatter(jj, h, buf, sem):
        pltpu.async_copy(buf.at[pl.ds(h * HC, HC)],
                         acc.at[rows2v.at[2 * jj + h]], sem, add=True)

    def _wait_scatters(jj, buf, sem):
        for h in range(2):
            pltpu.make_async_copy(buf.at[pl.ds(h * HC, HC)],
                                  acc.at[rows2v.at[2 * jj + h]], sem).wait()

    def _compute_half(jj, buf, h):
        def _edge(k, _):
            kk = jnp.full((16,), jj * C + k, jnp.int32)
            av = plsc.load_gather(vrv, [kk])
            bv = plsc.load_gather(viv, [kk])
            for j in range(4):
                slr = pl.ds(j * 16, 16)
                sli = pl.ds(H + j * 16, 16)
                xr = buf[k, slr]
                xi = buf[k, sli]
                buf[k, slr] = av * xr - bv * xi
                buf[k, sli] = av * xi + bv * xr
            return 0

        lax.fori_loop(h * HC, (h + 1) * HC, _edge, 0, unroll=4)

    def _half(jj, cur, nxt, sg_cur, sg_nxt, ss_cur, ss_nxt):
        @pl.when(jj >= 1)
        def _():
            _wait_scatters(jj - 1, nxt, ss_nxt)

        @pl.when(jj + 1 < NCHN)
        def _():
            _start_gather(jj + 1, nxt, sg_nxt)

        _wait_gather(jj, cur, sg_cur)
        _compute_half(jj, cur, 0)
        _start_scatter(jj, 0, cur, ss_cur)
        _compute_half(jj, cur, 1)
        _start_scatter(jj, 1, cur, ss_cur)

    def _super(s, _):
        pltpu.sync_copy(rows4_hbm.at[sid, s], rows2v)
        pltpu.sync_copy(cols4_hbm.at[sid, s], gidx2)
        pltpu.sync_copy(vr4_hbm.at[sid, s], vrv)
        pltpu.sync_copy(vi4_hbm.at[sid, s], viv)

        def _idx(jc, _):
            for i in range(C // 16):
                sl = pl.ds(i * 16, 16)
                gidx2[jc, sl] = gidx2[jc, sl] * 2 + cid
            return 0

        lax.fori_loop(0, NCHN, _idx, 0)

        _start_gather(0, xg0, sem_g0)

        def _pair(p, _):
            _half(2 * p, xg0, xg1, sem_g0, sem_g1, sem_s0, sem_s1)
            _half(2 * p + 1, xg1, xg0, sem_g1, sem_g0, sem_s1, sem_s0)
            return 0

        lax.fori_loop(0, NPAIR, _pair, 0)
        _wait_scatters(NCHN - 1, xg1, sem_s1)
        return 0

    lax.fori_loop(0, NSUP, _super, 0)
    plsc.subcore_barrier()

    for q in range(2):
        pltpu.sync_copy(
            acc.at[pl.ds(sid * RPT, RPT), pl.ds(q * H, H)],
            out_hbm.at[q, pl.ds(sid * RPT, RPT), pl.ds(cid * H, H)],
        )


@jax.jit
def kernel(L_values_real, L_values_imag, x_real, x_imag, L_indices):
    def _pad3(a):
        return jnp.pad(a.reshape(NS, NSUP, S), ((0, 0), (0, 0), (0, SP - S)))

    rows4 = _pad3(L_indices[0]).reshape(NS, NSUP, 2 * NCHN, HC)
    cols4 = _pad3(L_indices[1]).reshape(NS, NSUP, NCHN, C)
    vr4 = _pad3(L_values_real)
    vi4 = _pad3(L_values_imag)
    xc = jnp.concatenate(
        [x_real.reshape(2 * N, H), x_imag.reshape(2 * N, H)], axis=1)

    mesh = plsc.VectorSubcoreMesh(
        core_axis_name="c", subcore_axis_name="s", num_cores=NC,
        num_subcores=NS)
    f = pl.kernel(
        _sc_body,
        out_type=jax.ShapeDtypeStruct((2, N, D), jnp.float32),
        mesh=mesh,
        compiler_params=pltpu.CompilerParams(use_tc_tiling_on_sc=False,
                                             needs_layout_passes=False),
        scratch_types=[
            pltpu.VMEM_SHARED((N, D), jnp.float32),
            pltpu.VMEM((2 * NCHN, HC), jnp.int32),
            pltpu.VMEM((NCHN, C), jnp.int32),
            pltpu.VMEM((SP,), jnp.float32),
            pltpu.VMEM((SP,), jnp.float32),
            pltpu.VMEM((C, D), jnp.float32),
            pltpu.VMEM((C, D), jnp.float32),
            pltpu.SemaphoreType.DMA,
            pltpu.SemaphoreType.DMA,
            pltpu.SemaphoreType.DMA,
            pltpu.SemaphoreType.DMA,
        ],
    )
    return f(rows4, cols4, vr4, vi4, xc)


# speedup vs baseline: 11.7369x; 1.4714x over previous
import jax
import jax.numpy as jnp
from jax import lax
from jax.experimental import pallas as pl
from jax.experimental.pallas import tpu as pltpu
from jax.experimental.pallas import tpu_sc as plsc

N = 10000
E = 320000
D = 128
H = D // 2
NC = 2
NS = 16
EPW = E // NS
NSUP = 10
S = EPW // NSUP
C = 96
HC = C // 2
SP = 2016
NCHN = SP // C
NROT = NCHN // 3
RPT = N // NS
ZFULL = RPT // C
ZREM = RPT % C


def _sc_body(rows4_hbm, cols4_hbm, vr4_hbm, vi4_hbm, xc_hbm,
             out_hbm,
             acc, rows2, gidx2, vrv, viv,
             xg0, xg1, xg2, sg0, sg1, sg2, ss0, ss1, ss2):
    cid = lax.axis_index("c")
    sid = lax.axis_index("s")

    def _zero_buf(r, _):
        z = jnp.zeros((16,), jnp.float32)
        for j in range(8):
            xg0[r, pl.ds(j * 16, 16)] = z
        return 0

    lax.fori_loop(0, C, _zero_buf, 0)
    zbase = sid * RPT
    for t in range(ZFULL):
        pltpu.sync_copy(xg0, acc.at[pl.ds(zbase + t * C, C)])
    pltpu.sync_copy(xg0.at[pl.ds(0, ZREM)],
                    acc.at[pl.ds(zbase + ZFULL * C, ZREM)])
    plsc.subcore_barrier()

    def _start_gather(jj, buf, sem):
        pltpu.async_copy(xc_hbm.at[gidx2.at[jj]], buf, sem)

    def _wait_gather(jj, buf, sem):
        pltpu.make_async_copy(xc_hbm.at[gidx2.at[jj]], buf, sem).wait()

    def _start_scatter(jj, buf, sem):
        pltpu.async_copy(buf, acc.at[rows2.at[jj]], sem, add=True)

    def _wait_scatter(jj, buf, sem):
        pltpu.make_async_copy(buf, acc.at[rows2.at[jj]], sem).wait()

    def _compute_half(jj, buf, h):
        def _edge(k, _):
            kk = jnp.full((16,), jj * C + k, jnp.int32)
            av = plsc.load_gather(vrv, [kk])
            bv = plsc.load_gather(viv, [kk])
            for j in range(4):
                slr = pl.ds(j * 16, 16)
                sli = pl.ds(H + j * 16, 16)
                xr = buf[k, slr]
                xi = buf[k, sli]
                buf[k, slr] = av * xr - bv * xi
                buf[k, sli] = av * xi + bv * xr
            return 0

        lax.fori_loop(h * HC, (h + 1) * HC, _edge, 0, unroll=4)

    def _third(jj, buf_a, buf_c, sg_a, sg_c, ss_a, ss_c):
        _wait_gather(jj, buf_a, sg_a)
        _compute_half(jj, buf_a, 0)

        @pl.when(jj >= 1)
        def _():
            _wait_scatter(jj - 1, buf_c, ss_c)

        @pl.when(jj + 2 < NCHN)
        def _():
            _start_gather(jj + 2, buf_c, sg_c)

        _compute_half(jj, buf_a, 1)
        _start_scatter(jj, buf_a, ss_a)

    def _super(s, _):
        pltpu.sync_copy(rows4_hbm.at[sid, s], rows2)
        pltpu.sync_copy(cols4_hbm.at[sid, s], gidx2)
        pltpu.sync_copy(vr4_hbm.at[sid, s], vrv)
        pltpu.sync_copy(vi4_hbm.at[sid, s], viv)

        def _idx(jc, _):
            for i in range(C // 16):
                sl = pl.ds(i * 16, 16)
                gidx2[jc, sl] = gidx2[jc, sl] * 2 + cid
            return 0

        lax.fori_loop(0, NCHN, _idx, 0)

        _start_gather(0, xg0, sg0)
        _start_gather(1, xg1, sg1)

        def _rot(r, _):
            _third(3 * r, xg0, xg2, sg0, sg2, ss0, ss2)
            _third(3 * r + 1, xg1, xg0, sg1, sg0, ss1, ss0)
            _third(3 * r + 2, xg2, xg1, sg2, sg1, ss2, ss1)
            return 0

        lax.fori_loop(0, NROT, _rot, 0)
        _wait_scatter(NCHN - 1, xg2, ss2)
        return 0

    lax.fori_loop(0, NSUP, _super, 0)
    plsc.subcore_barrier()

    for q in range(2):
        pltpu.sync_copy(
            acc.at[pl.ds(sid * RPT, RPT), pl.ds(q * H, H)],
            out_hbm.at[q, pl.ds(sid * RPT, RPT), pl.ds(cid * H, H)],
        )


@jax.jit
def kernel(L_values_real, L_values_imag, x_real, x_imag, L_indices):
    def _pad3(a):
        return jnp.pad(a.reshape(NS, NSUP, S), ((0, 0), (0, 0), (0, SP - S)))

    rows4 = _pad3(L_indices[0]).reshape(NS, NSUP, NCHN, C)
    cols4 = _pad3(L_indices[1]).reshape(NS, NSUP, NCHN, C)
    vr4 = _pad3(L_values_real)
    vi4 = _pad3(L_values_imag)
    xc = jnp.concatenate(
        [x_real.reshape(2 * N, H), x_imag.reshape(2 * N, H)], axis=1)

    mesh = plsc.VectorSubcoreMesh(
        core_axis_name="c", subcore_axis_name="s", num_cores=NC,
        num_subcores=NS)
    f = pl.kernel(
        _sc_body,
        out_type=jax.ShapeDtypeStruct((2, N, D), jnp.float32),
        mesh=mesh,
        compiler_params=pltpu.CompilerParams(use_tc_tiling_on_sc=False,
                                             needs_layout_passes=False),
        scratch_types=[
            pltpu.VMEM_SHARED((N, D), jnp.float32),
            pltpu.VMEM((NCHN, C), jnp.int32),
            pltpu.VMEM((NCHN, C), jnp.int32),
            pltpu.VMEM((SP,), jnp.float32),
            pltpu.VMEM((SP,), jnp.float32),
            pltpu.VMEM((C, D), jnp.float32),
            pltpu.VMEM((C, D), jnp.float32),
            pltpu.VMEM((C, D), jnp.float32),
            pltpu.SemaphoreType.DMA,
            pltpu.SemaphoreType.DMA,
            pltpu.SemaphoreType.DMA,
            pltpu.SemaphoreType.DMA,
            pltpu.SemaphoreType.DMA,
            pltpu.SemaphoreType.DMA,
        ],
    )
    return f(rows4, cols4, vr4, vi4, xc)
